# baseline (speedup 1.0000x reference)
import functools

import jax
import jax.numpy as jnp
from jax import lax
from jax.experimental import pallas as pl
from jax.experimental.pallas import tpu as pltpu
from jax.experimental.pallas import tpu_sc as plsc

_N = 10000
_E = 320000
_D = 128
_NC = 2
_NS = 16
_NW = _NC * _NS
_EPT = 10240
_EPAD = _NW * _EPT
_K = 80
_NCHUNK = _EPT // _K
_NP = 10240
_RPT = _NP // _NS


def _sc_segment_sum(h, src, dst, zrows):
  mesh = plsc.VectorSubcoreMesh(core_axis_name="c", subcore_axis_name="s",
                                num_cores=_NC, num_subcores=_NS)

  @functools.partial(
      pl.kernel,
      out_type=(jax.ShapeDtypeStruct((_NC, _NP, _D), jnp.float32),
                jax.ShapeDtypeStruct((_NC, _NP), jnp.float32)),
      mesh=mesh,
      scratch_types=[
          pltpu.VMEM_SHARED((_NP, _D), jnp.float32),
          pltpu.VMEM_SHARED((_NP,), jnp.float32),
      ] + [pltpu.VMEM((_K,), jnp.int32)] * 8 + [
          pltpu.VMEM((_K, _D), jnp.float32),
          pltpu.VMEM((_K, _D), jnp.float32),
          pltpu.VMEM((_K, _D), jnp.float32),
          pltpu.VMEM((_K, _D), jnp.float32),
          pltpu.VMEM((_K,), jnp.float32),
          pltpu.VMEM((_RPT,), jnp.float32),
      ] + [pltpu.SemaphoreType.DMA] * 12,
      compiler_params=pltpu.CompilerParams(needs_layout_passes=False),
  )
  def seg_sum(h_hbm, src_hbm, dst_hbm, zrows_hbm,
              agg_out, cnt_out, agg_sp, cnt_sp,
              idx_s_a, idx_s_b, idx_s_c, idx_s_d,
              idx_d_a, idx_d_b, idx_d_c, idx_d_d,
              rows_a, rows_b, rows_c, rows_d, ones_v, cbuf,
              sem_sa, sem_sb, sem_sc, sem_sd,
              sem_da, sem_db, sem_dc, sem_dd,
              sem_ra, sem_rb, sem_rc, sem_rd):
    c = lax.axis_index("c")
    s = lax.axis_index("s")
    r0 = s * _RPT
    base = (c * _NS + s) * _EPT
    zeros16 = jnp.zeros((16,), jnp.float32)
    ones16 = jnp.ones((16,), jnp.float32)
    for j in range(_K // 16):
      ones_v[pl.ds(j * 16, 16)] = ones16
    for j in range(_RPT // 16):
      cbuf[pl.ds(j * 16, 16)] = zeros16
    pltpu.sync_copy(cbuf, cnt_sp.at[pl.ds(r0, _RPT)])
    pltpu.sync_copy(zrows_hbm, rows_a)
    for j in range(_RPT // _K):
      pltpu.sync_copy(rows_a, agg_sp.at[pl.ds(r0 + j * _K, _K)])
    plsc.subcore_barrier()

    bufs = ((idx_s_a, idx_d_a, rows_a, sem_sa, sem_da, sem_ra),
            (idx_s_b, idx_d_b, rows_b, sem_sb, sem_db, sem_rb),
            (idx_s_c, idx_d_c, rows_c, sem_sc, sem_dc, sem_rc),
            (idx_s_d, idx_d_d, rows_d, sem_sd, sem_dd, sem_rd))
    _NB = len(bufs)

    def fetch_idx(g, bi):
      idx_s_buf, idx_d_buf, _, sem_s, sem_d, _ = bufs[bi]
      pltpu.async_copy(src_hbm.at[pl.ds(base + g * _K, _K)], idx_s_buf,
                       sem_s)
      pltpu.async_copy(dst_hbm.at[pl.ds(base + g * _K, _K)], idx_d_buf,
                       sem_d)

    def start_gather(bi):
      idx_s_buf, _, rows_buf, sem_s, _, sem_r = bufs[bi]
      pltpu.make_async_copy(src_hbm.at[pl.ds(0, _K)], idx_s_buf,
                            sem_s).wait()
      pltpu.async_copy(h_hbm.at[idx_s_buf], rows_buf, sem_r)

    def process(bi):
      _, idx_d_buf, rows_buf, _, sem_d, sem_r = bufs[bi]
      pltpu.make_async_copy(h_hbm.at[pl.ds(0, _K)], rows_buf, sem_r).wait()
      pltpu.make_async_copy(dst_hbm.at[pl.ds(0, _K)], idx_d_buf,
                            sem_d).wait()
      pltpu.sync_copy(rows_buf, agg_sp.at[idx_d_buf], add=True)
      pltpu.sync_copy(ones_v, cnt_sp.at[idx_d_buf], add=True)

    for g0 in range(_NB):
      fetch_idx(g0, g0)
    start_gather(0)
    start_gather(1)
    start_gather(2)

    def body(t, carry):
      for bi in range(_NB):
        g = _NB * t + bi
        process(bi)
        pg = g + _NB

        @pl.when(pg < _NCHUNK)
        def _():
          fetch_idx(pg, bi)

        ng = g + 3

        @pl.when(ng < _NCHUNK)
        def _():
          start_gather((bi + 3) % _NB)

      return carry

    lax.fori_loop(0, _NCHUNK // _NB, body, 0)
    plsc.subcore_barrier()

    for j in range(_RPT // _K):
      pltpu.sync_copy(agg_sp.at[pl.ds(r0 + j * _K, _K)], rows_a)
      pltpu.sync_copy(rows_a, agg_out.at[c, pl.ds(r0 + j * _K, _K)])
    pltpu.sync_copy(cnt_sp.at[pl.ds(r0, _RPT)], cbuf)
    pltpu.sync_copy(cbuf, cnt_out.at[c, pl.ds(r0, _RPT)])

  return seg_sum(h, src, dst, zrows)


def _tc_self(h, Wr):
  def body(h_ref, wr_ref, out_ref):
    out_ref[...] = lax.dot_general(h_ref[...], wr_ref[...],
                                   (((1,), (1,)), ((), ())),
                                   preferred_element_type=jnp.float32)

  return pl.pallas_call(
      body,
      out_shape=jax.ShapeDtypeStruct((_N, _D), jnp.float32),
  )(h, Wr)


def _tc_merge(p, cntp, hr, Wl, bl, gw, gb, gms):
  def body(p_ref, cnt_ref, hr_ref, wl_ref, bl_ref, gw_ref, gb_ref,
           gms_ref, out_ref):
    agg = p_ref[0, :_N] + p_ref[1, :_N]
    cnt_row = jnp.sum(cnt_ref[...], axis=0, keepdims=True)
    cnt = jnp.transpose(cnt_row[:, :_N])
    agg = agg / jnp.maximum(cnt, 1.0)
    z = (lax.dot_general(agg, wl_ref[...], (((1,), (1,)), ((), ())),
                         preferred_element_type=jnp.float32)
         + bl_ref[...] + hr_ref[...])
    z = jnp.maximum(z, 0.0)
    mean = jnp.mean(z, axis=0, keepdims=True)
    out = z - mean * gms_ref[...]
    var = jnp.mean(out * out, axis=0, keepdims=True)
    out = out * lax.rsqrt(var + 1e-5)
    out_ref[...] = out * gw_ref[...] + gb_ref[...]

  return pl.pallas_call(
      body,
      out_shape=jax.ShapeDtypeStruct((_N, _D), jnp.float32),
  )(p, cntp, hr, Wl, bl, gw, gb, gms)


def kernel(x, edge_index, W1l, b1l, W1r, W2l, b2l, W2r,
           gn_weight, gn_bias, gn_mean_scale):
  src = edge_index[0].astype(jnp.int32)
  dst = edge_index[1].astype(jnp.int32)
  npad = _EPAD - _E
  pad_src = jnp.arange(npad, dtype=jnp.int32) % _N
  src = jnp.concatenate([src, pad_src])
  pad_dst = _N + (jnp.arange(npad, dtype=jnp.int32) % (_NP - _N))
  dst = jnp.concatenate([dst, pad_dst])
  zrows = jnp.zeros((_K, _D), jnp.float32)
  gw = gn_weight.reshape(1, _D)
  gb = gn_bias.reshape(1, _D)
  gms = gn_mean_scale.reshape(1, _D)
  h = x
  for Wl, bl, Wr in ((W1l, b1l, W1r), (W2l, b2l, W2r)):
    hr = _tc_self(h, Wr)
    p, cntp = _sc_segment_sum(h, src, dst, zrows)
    h = _tc_merge(p, cntp, hr, Wl, bl.reshape(1, _D), gw, gb, gms)
  return h


# speedup vs baseline: 1.0124x; 1.0124x over previous
import functools

import jax
import jax.numpy as jnp
from jax import lax
from jax.experimental import pallas as pl
from jax.experimental.pallas import tpu as pltpu
from jax.experimental.pallas import tpu_sc as plsc

_N = 10000
_E = 320000
_D = 128
_NC = 2
_NS = 16
_NW = _NC * _NS
_EPT = _E // _NW
_K = 80
_NCHUNK = _EPT // _K
_NP = 10240
_RPT = _NP // _NS


def _sc_segment_sum(h, src, dst, zrows):
  mesh = plsc.VectorSubcoreMesh(core_axis_name="c", subcore_axis_name="s",
                                num_cores=_NC, num_subcores=_NS)

  @functools.partial(
      pl.kernel,
      out_type=(jax.ShapeDtypeStruct((_NC, _NP, _D), jnp.float32),
                jax.ShapeDtypeStruct((_NC, _NP), jnp.float32)),
      mesh=mesh,
      scratch_types=[
          pltpu.VMEM_SHARED((_NP, _D), jnp.float32),
          pltpu.VMEM_SHARED((_NP,), jnp.float32),
          pltpu.VMEM((_EPT,), jnp.int32),
          pltpu.VMEM((_K,), jnp.int32),
          pltpu.VMEM((_K,), jnp.int32),
          pltpu.VMEM((_K,), jnp.int32),
          pltpu.VMEM((_K, _D), jnp.float32),
          pltpu.VMEM((_K, _D), jnp.float32),
          pltpu.VMEM((_K, _D), jnp.float32),
          pltpu.VMEM((_K,), jnp.float32),
          pltpu.VMEM((_RPT,), jnp.float32),
          pltpu.SemaphoreType.DMA,
          pltpu.SemaphoreType.DMA,
          pltpu.SemaphoreType.DMA,
          pltpu.SemaphoreType.DMA,
          pltpu.SemaphoreType.DMA,
          pltpu.SemaphoreType.DMA,
      ],
      compiler_params=pltpu.CompilerParams(needs_layout_passes=False),
  )
  def seg_sum(h_hbm, src_hbm, dst_hbm, zrows_hbm,
              agg_out, cnt_out, agg_sp, cnt_sp, idx_all_s,
              idx_d_a, idx_d_b, idx_d_c,
              rows_a, rows_b, rows_c, ones_v, cbuf,
              sem_ra, sem_rb, sem_rc, sem_ia, sem_ib, sem_ic):
    c = lax.axis_index("c")
    s = lax.axis_index("s")
    r0 = s * _RPT
    base = (c * _NS + s) * _EPT
    pltpu.sync_copy(src_hbm.at[pl.ds(base, _EPT)], idx_all_s)
    zeros16 = jnp.zeros((16,), jnp.float32)
    ones16 = jnp.ones((16,), jnp.float32)
    for j in range(_K // 16):
      ones_v[pl.ds(j * 16, 16)] = ones16
    for j in range(_RPT // 16):
      cbuf[pl.ds(j * 16, 16)] = zeros16
    pltpu.sync_copy(cbuf, cnt_sp.at[pl.ds(r0, _RPT)])
    pltpu.sync_copy(zrows_hbm, rows_a)
    for j in range(_RPT // _K):
      pltpu.sync_copy(rows_a, agg_sp.at[pl.ds(r0 + j * _K, _K)])
    plsc.subcore_barrier()

    def fetch(g, idx_d_buf, rows_buf, sem_r, sem_i):
      pltpu.async_copy(dst_hbm.at[pl.ds(base + g * _K, _K)], idx_d_buf,
                       sem_i)
      pltpu.async_copy(h_hbm.at[idx_all_s.at[pl.ds(g * _K, _K)]],
                       rows_buf, sem_r)

    def wait_chunk(idx_d_buf, rows_buf, sem_r, sem_i):
      pltpu.make_async_copy(dst_hbm.at[pl.ds(0, _K)], idx_d_buf,
                            sem_i).wait()
      pltpu.make_async_copy(h_hbm.at[pl.ds(0, _K)], rows_buf, sem_r).wait()

    bufs = ((idx_d_a, rows_a, sem_ra, sem_ia),
            (idx_d_b, rows_b, sem_rb, sem_ib),
            (idx_d_c, rows_c, sem_rc, sem_ic))
    _NB = len(bufs)
    for g0 in range(_NB):
      fetch(g0, *bufs[g0])

    def process(g, idx_d_buf, rows_buf, sem_r, sem_i):
      wait_chunk(idx_d_buf, rows_buf, sem_r, sem_i)
      pltpu.sync_copy(rows_buf, agg_sp.at[idx_d_buf], add=True)
      pltpu.sync_copy(ones_v, cnt_sp.at[idx_d_buf], add=True)

    def body(t, carry):
      for bi in range(_NB):
        g = _NB * t + bi
        process(g, *bufs[bi])
        pg = g + _NB

        @pl.when(pg < _NCHUNK)
        def _():
          fetch(pg, *bufs[bi])

      return carry

    lax.fori_loop(0, _NCHUNK // _NB, body, 0)
    for g in range(_NCHUNK - _NCHUNK % _NB, _NCHUNK):
      process(g, *bufs[g % _NB])
    plsc.subcore_barrier()

    for j in range(_RPT // _K):
      pltpu.sync_copy(agg_sp.at[pl.ds(r0 + j * _K, _K)], rows_a)
      pltpu.sync_copy(rows_a, agg_out.at[c, pl.ds(r0 + j * _K, _K)])
    pltpu.sync_copy(cnt_sp.at[pl.ds(r0, _RPT)], cbuf)
    pltpu.sync_copy(cbuf, cnt_out.at[c, pl.ds(r0, _RPT)])

  return seg_sum(h, src, dst, zrows)


def _tc_self(h, Wr):
  def body(h_ref, wr_ref, out_ref):
    out_ref[...] = lax.dot_general(h_ref[...], wr_ref[...],
                                   (((1,), (1,)), ((), ())),
                                   preferred_element_type=jnp.float32)

  return pl.pallas_call(
      body,
      out_shape=jax.ShapeDtypeStruct((_N, _D), jnp.float32),
  )(h, Wr)


def _tc_merge(p, cntp, hr, Wl, bl, gw, gb, gms):
  def body(p_ref, cnt_ref, hr_ref, wl_ref, bl_ref, gw_ref, gb_ref,
           gms_ref, out_ref):
    agg = p_ref[0, :_N] + p_ref[1, :_N]
    cnt_row = jnp.sum(cnt_ref[...], axis=0, keepdims=True)
    cnt = jnp.transpose(cnt_row[:, :_N])
    agg = agg / jnp.maximum(cnt, 1.0)
    z = (lax.dot_general(agg, wl_ref[...], (((1,), (1,)), ((), ())),
                         preferred_element_type=jnp.float32)
         + bl_ref[...] + hr_ref[...])
    z = jnp.maximum(z, 0.0)
    mean = jnp.mean(z, axis=0, keepdims=True)
    out = z - mean * gms_ref[...]
    var = jnp.mean(out * out, axis=0, keepdims=True)
    out = out * lax.rsqrt(var + 1e-5)
    out_ref[...] = out * gw_ref[...] + gb_ref[...]

  return pl.pallas_call(
      body,
      out_shape=jax.ShapeDtypeStruct((_N, _D), jnp.float32),
  )(p, cntp, hr, Wl, bl, gw, gb, gms)


def kernel(x, edge_index, W1l, b1l, W1r, W2l, b2l, W2r,
           gn_weight, gn_bias, gn_mean_scale):
  src = edge_index[0].astype(jnp.int32)
  dst = edge_index[1].astype(jnp.int32)
  zrows = jnp.zeros((_K, _D), jnp.float32)
  gw = gn_weight.reshape(1, _D)
  gb = gn_bias.reshape(1, _D)
  gms = gn_mean_scale.reshape(1, _D)
  h = x
  for Wl, bl, Wr in ((W1l, b1l, W1r), (W2l, b2l, W2r)):
    hr = _tc_self(h, Wr)
    p, cntp = _sc_segment_sum(h, src, dst, zrows)
    h = _tc_merge(p, cntp, hr, Wl, bl.reshape(1, _D), gw, gb, gms)
  return h
